# Initial kernel scaffold; baseline (speedup 1.0000x reference)
import jax
import jax.numpy as jnp
from jax.experimental import pallas as pl


def kernel(pred, y):
    raise NotImplementedError("write your pallas kernel here")


# speedup vs baseline: 22.7467x; 22.7467x over previous
import jax
import jax.numpy as jnp
from jax import lax
from jax.experimental import pallas as pl
from jax.experimental.pallas import tpu as pltpu
from jax.experimental.pallas import tpu_sc as plsc

B, C = 16384, 1000
NW = 32
RPW = B // NW
CH = 32
NCHUNK = RPW // CH
GPC = CH // 16


def _cw_body(pred_hbm, y_hbm, out_hbm, buf0, buf1, y_v, out_v, sem0, sem1):
    cid = lax.axis_index("c")
    sid = lax.axis_index("s")
    wid = sid * 2 + cid
    row0 = wid * RPW
    pltpu.sync_copy(y_hbm.at[pl.ds(row0, RPW)], y_v)

    bufs = [buf0, buf1]
    sems = [sem0, sem1]
    neg_inf = jnp.full((16,), -jnp.inf, jnp.float32)
    lane = lax.iota(jnp.int32, 16)

    copies = [None, None]
    copies[0] = pltpu.async_copy(
        pred_hbm.at[pl.ds(row0 * C, CH * C)], bufs[0], sems[0])
    for c in range(NCHUNK):
        if c + 1 < NCHUNK:
            nb = (c + 1) % 2
            copies[nb] = pltpu.async_copy(
                pred_hbm.at[pl.ds((row0 + (c + 1) * CH) * C, CH * C)],
                bufs[nb], sems[nb])
        cb = c % 2
        copies[cb].wait()
        buf = bufs[cb]
        for g in range(GPC):
            gg = c * GPC + g
            yv = y_v[pl.ds(gg * 16, 16)]
            rowbase = (g * 16 + lane) * C
            tgt = rowbase + yv
            class_pred = plsc.load_gather(buf, [tgt])
            plsc.store_scatter(buf, [tgt], neg_inf)

            def body(i, carry):
                a0, a1, cv = carry
                for _ in range(4):
                    v0 = plsc.load_gather(buf, [cv])
                    v1 = plsc.load_gather(buf, [cv + 1])
                    a0 = jnp.maximum(a0, v0)
                    a1 = jnp.maximum(a1, v1)
                    cv = cv + 2
                return a0, a1, cv

            acc0, acc1, _ = lax.fori_loop(
                0, C // 8, body, (neg_inf, neg_inf, rowbase))
            loss = jnp.maximum(acc0, acc1) - class_pred
            out_v[pl.ds(gg * 16, 16)] = loss

    pltpu.sync_copy(out_v, out_hbm.at[pl.ds(row0, RPW)])


_run = pl.kernel(
    _cw_body,
    out_type=jax.ShapeDtypeStruct((B,), jnp.float32),
    mesh=plsc.VectorSubcoreMesh(core_axis_name="c", subcore_axis_name="s"),
    scratch_types=[
        pltpu.VMEM((CH * C,), jnp.float32),
        pltpu.VMEM((CH * C,), jnp.float32),
        pltpu.VMEM((RPW,), jnp.int32),
        pltpu.VMEM((RPW,), jnp.float32),
        pltpu.SemaphoreType.DMA,
        pltpu.SemaphoreType.DMA,
    ],
    compiler_params=pltpu.CompilerParams(needs_layout_passes=False),
)


@jax.jit
def kernel(pred, y):
    return _run(jnp.reshape(pred, (-1,)), y.astype(jnp.int32))
